# baseline (speedup 1.0000x reference)
import functools

import jax
import jax.numpy as jnp
from jax import lax
from jax.experimental import pallas as pl
from jax.experimental.pallas import tpu as pltpu
from jax.experimental.pallas import tpu_sc as plsc

B = 16384
D = 16
OUTDIM = 64
NE = 1000000
EPS = 1e-5

NUM_CORES = 2
NUM_SUBCORES = 16
NW = NUM_CORES * NUM_SUBCORES
BPW = B // NW

CW = 2048
NCHUNK = 489
TAILW = 640
CPT = 16
EOUT_ROWS = B + 16


ABLK = 131072
NABLK = (NE + ABLK - 1) // ABLK


def _alpha_body(alpha_ref, alin_ref):
    alin_ref[...] = alpha_ref[0, :]


_tc_detile_alpha = pl.pallas_call(
    _alpha_body,
    grid=(NABLK,),
    in_specs=[pl.BlockSpec((1, ABLK), lambda j: (0, j))],
    out_specs=pl.BlockSpec((ABLK,), lambda j: (j,)),
    out_shape=jax.ShapeDtypeStruct((NABLK * ABLK,), jnp.float32),
)


def _sc_body(emb_hbm, alpha_hbm, idx_hbm, e_out, a_out,
             idx_piece, idx_a, hits_col, hits_pos, slab, stage, pos16_v,
             arow_v, sem_a, sem_s):
    wid = lax.axis_index("s") * NUM_CORES + lax.axis_index("c")
    lanes = lax.iota(jnp.int32, 16)
    ones16 = jnp.full((16,), 1, jnp.int32)

    base = wid * BPW
    pltpu.sync_copy(idx_hbm.at[pl.ds(base, BPW)], idx_a)
    cp_a = pltpu.async_copy(alpha_hbm.at[idx_a], arow_v, sem_a)

    def _prep_piece(piece, off):
        pltpu.sync_copy(idx_hbm.at[pl.ds(piece * BPW, BPW)], idx_piece)

        def _prep_group(g, off_):
            v = idx_piece[pl.ds(g * 16, 16)]
            cid = lax.shift_right_logical(v, 11)
            m = (cid & 31) == wid
            ranks = plsc.cumsum(jnp.where(m, 1, 0).astype(jnp.int32))
            slots = ranks - 1 + off_
            gpos = piece * BPW + g * 16 + lanes
            plsc.store_scatter(hits_col, [slots], v, mask=m)
            plsc.store_scatter(hits_pos, [slots], gpos, mask=m)
            return off_ + lax.reduce_max(ranks, (0,))

        return lax.fori_loop(0, BPW // 16, _prep_group, off)

    cnt = lax.fori_loop(0, NW, _prep_piece, jnp.int32(0))
    cp_a.wait()
    pltpu.sync_copy(arow_v, a_out.at[pl.ds(base, BPW)])

    def _chunk(cc, carry):
        c = cc * 32 + wid

        def _process(width):
            pltpu.sync_copy(
                emb_hbm.at[pl.ds(0, 8), pl.ds(c * CW, width)],
                slab.at[pl.ds(0, 8), pl.ds(0, width)])
            pltpu.sync_copy(
                emb_hbm.at[pl.ds(8, 8), pl.ds(c * CW, width)],
                slab.at[pl.ds(8, 8), pl.ds(0, width)])

            ngq = lax.shift_right_logical(cnt + 15, 4)

            def _batch(b, carry_):
                hv = hits_col[pl.ds(b * 16, 16)]
                hp = hits_pos[pl.ds(b * 16, 16)]
                valid = (b * 16 + lanes) < cnt
                m2 = valid & (lax.shift_right_logical(hv, 11) == c)
                nhit = lax.reduce_max(
                    plsc.cumsum(jnp.where(m2, 1, 0).astype(jnp.int32)), (0,))

                @pl.when(nhit > 0)
                def _():
                    pos = jnp.where(m2, hp, B + lanes)
                    pos16_v[...] = pos
                    local = (hv - c * CW) & (CW - 1)
                    for j in range(16):
                        lj = plsc.load_gather(hits_col, [ones16 * (b * 16 + j)])
                        ljl = (lj - c * CW) & (CW - 1)
                        row = plsc.load_gather(slab, [lanes, ljl])
                        stage[j, pl.ds(0, 16)] = row
                    pltpu.async_copy(stage, e_out.at[pos16_v], sem_s).wait()
                return carry_

            lax.fori_loop(0, ngq, _batch, 0)

        @pl.when(c < 488)
        def _():
            _process(CW)

        @pl.when(c == 488)
        def _():
            _process(TAILW)

        return carry

    lax.fori_loop(0, CPT, _chunk, 0)


@functools.lru_cache(maxsize=None)
def _sc_gather():
    return pl.kernel(
        _sc_body,
        out_type=(
            jax.ShapeDtypeStruct((EOUT_ROWS, 128), jnp.float32),
            jax.ShapeDtypeStruct((B,), jnp.float32),
        ),
        mesh=plsc.VectorSubcoreMesh(
            core_axis_name="c", subcore_axis_name="s",
            num_cores=NUM_CORES, num_subcores=NUM_SUBCORES,
        ),
        scratch_types=[
            pltpu.VMEM((BPW,), jnp.int32),
            pltpu.VMEM((BPW,), jnp.int32),
            pltpu.VMEM((B,), jnp.int32),
            pltpu.VMEM((B,), jnp.int32),
            pltpu.VMEM((16, CW), jnp.float32),
            pltpu.VMEM((16, 128), jnp.float32),
            pltpu.VMEM((16,), jnp.int32),
            pltpu.VMEM((BPW,), jnp.float32),
            pltpu.SemaphoreType.DMA,
            pltpu.SemaphoreType.DMA,
        ],
        compiler_params=pltpu.CompilerParams(use_tc_tiling_on_sc=True,
                                             needs_layout_passes=False),
    )


def _tc_body(e_ref, wt_ref, g_ref, bb_ref, out_ref):
    e = e_ref[pl.ds(0, B), pl.ds(0, D)]
    wt = wt_ref[...]
    inv_b = 1.0 / B
    ones_b = jnp.ones((B, 1), jnp.float32)
    ones_d = jnp.ones((D, 1), jnp.float32)
    mean_e = lax.dot_general(e, ones_b, (((0,), (0,)), ((), ())),
                             preferred_element_type=jnp.float32) * inv_b
    smom = lax.dot_general(e, e, (((0,), (0,)), ((), ())),
                           preferred_element_type=jnp.float32) * inv_b
    m_t = lax.dot_general(wt, mean_e, (((0,), (0,)), ((), ())))
    p = lax.dot_general(smom, wt, (((1,), (0,)), ((), ())))
    ey2_t = lax.dot_general(wt * p, ones_d, (((0,), (0,)), ((), ())))
    var_t = ey2_t - m_t * m_t
    scale_t = g_ref[...] * lax.rsqrt(var_t + EPS)
    shift_t = bb_ref[...] - m_t * scale_t
    y_t = lax.dot_general(wt, e, (((0,), (1,)), ((), ())),
                          preferred_element_type=jnp.float32)
    out_ref[...] = y_t * scale_t + shift_t


_tc_transform = pl.pallas_call(
    _tc_body,
    out_shape=jax.ShapeDtypeStruct((OUTDIM, B), jnp.float32),
)


@jax.jit
def kernel(x, emb_table, alpha_table, W, b, gamma, beta):
    del b
    alpha_lin = _tc_detile_alpha(alpha_table.T)
    e_pad, alpha = _sc_gather()(emb_table.T, alpha_lin, x)
    y_t = _tc_transform(e_pad, W.T, gamma.reshape(OUTDIM, 1),
                        beta.reshape(OUTDIM, 1))
    return (y_t.T, alpha.reshape(B, 1))


# speedup vs baseline: 3.4264x; 3.4264x over previous
import functools

import jax
import jax.numpy as jnp
from jax import lax
from jax.experimental import pallas as pl
from jax.experimental.pallas import tpu as pltpu
from jax.experimental.pallas import tpu_sc as plsc

B = 16384
D = 16
OUTDIM = 64
NE = 1000000
EPS = 1e-5

NUM_CORES = 2
NUM_SUBCORES = 16
NW = NUM_CORES * NUM_SUBCORES
BPW = B // NW

CW = 2048
TAILW = 640
CPT = 16
IPW = 4096
NPIECE = B // IPW
STAGE_ROWS = 128
EOUT_ROWS = B + STAGE_ROWS


ABLK = 131072
NABLK = (NE + ABLK - 1) // ABLK


def _alpha_body(alpha_ref, alin_ref):
    alin_ref[...] = alpha_ref[0, :]


_tc_detile_alpha = pl.pallas_call(
    _alpha_body,
    grid=(NABLK,),
    in_specs=[pl.BlockSpec((1, ABLK), lambda j: (0, j))],
    out_specs=pl.BlockSpec((ABLK,), lambda j: (j,)),
    out_shape=jax.ShapeDtypeStruct((NABLK * ABLK,), jnp.float32),
)


def _sc_body(emb_hbm, alpha_hbm, idx_hbm, e_out, a_out,
             idx_piece, idx_a, hits_col, hits_pos, slab0, slab1,
             stage, pos_stage, arow_v, sem_a, sem_sl0, sem_sl1, sem_sc):
    wid = lax.axis_index("s") * NUM_CORES + lax.axis_index("c")
    lanes = lax.iota(jnp.int32, 16)
    slabs = [slab0, slab1]
    slab_sems = [sem_sl0, sem_sl1]

    base = wid * BPW
    pltpu.sync_copy(idx_hbm.at[pl.ds(base, BPW)], idx_a)
    cp_a = pltpu.async_copy(alpha_hbm.at[idx_a], arow_v, sem_a)

    def _start_slab(c, buf):
        def go(width):
            pltpu.async_copy(
                emb_hbm.at[pl.ds(0, 8), pl.ds(c * CW, width)],
                slabs[buf].at[pl.ds(0, 8), pl.ds(0, width)], slab_sems[buf])
            pltpu.async_copy(
                emb_hbm.at[pl.ds(8, 8), pl.ds(c * CW, width)],
                slabs[buf].at[pl.ds(8, 8), pl.ds(0, width)], slab_sems[buf])
        return go

    def _wait_slab(c, buf):
        def go(width):
            pltpu.make_async_copy(
                emb_hbm.at[pl.ds(0, 8), pl.ds(c * CW, width)],
                slabs[buf].at[pl.ds(0, 8), pl.ds(0, width)],
                slab_sems[buf]).wait()
            pltpu.make_async_copy(
                emb_hbm.at[pl.ds(8, 8), pl.ds(c * CW, width)],
                slabs[buf].at[pl.ds(8, 8), pl.ds(0, width)],
                slab_sems[buf]).wait()
        return go

    _start_slab(wid, 0)(CW)

    def _prep_piece(piece, off):
        pltpu.sync_copy(idx_hbm.at[pl.ds(piece * IPW, IPW)], idx_piece)

        def _prep_group(g, off_):
            v = idx_piece[pl.ds(g * 16, 16)]
            cid = lax.shift_right_logical(v, 11)
            m = (cid & 31) == wid
            ranks = plsc.cumsum(jnp.where(m, 1, 0).astype(jnp.int32))
            slots = ranks - 1 + off_
            gpos = piece * IPW + g * 16 + lanes
            plsc.store_scatter(hits_col, [slots], v, mask=m)
            plsc.store_scatter(hits_pos, [slots], gpos, mask=m)
            return off_ + lax.reduce_max(ranks, (0,))

        return lax.fori_loop(0, IPW // 16, _prep_group, off)

    cnt = lax.fori_loop(0, NPIECE, _prep_piece, jnp.int32(0))
    cp_a.wait()
    pltpu.sync_copy(arow_v, a_out.at[pl.ds(base, BPW)])

    for g in range(STAGE_ROWS // 16):
        pos_stage[pl.ds(g * 16, 16)] = B + g * 16 + lanes

    ngq = lax.shift_right_logical(cnt + 15, 4)
    o = jnp.int32(0)

    for cc in range(CPT):
        c = cc * 32 + wid
        buf = cc & 1

        @pl.when(c < 488)
        def _(c=c, buf=buf):
            _wait_slab(c, buf)(CW)

        @pl.when(c == 488)
        def _(c=c, buf=buf):
            _wait_slab(c, buf)(TAILW)

        if cc + 1 < CPT:
            cn = (cc + 1) * 32 + wid
            nbuf = (cc + 1) & 1

            @pl.when(cn < 488)
            def _(cn=cn, nbuf=nbuf):
                _start_slab(cn, nbuf)(CW)

            @pl.when(cn == 488)
            def _(cn=cn, nbuf=nbuf):
                _start_slab(cn, nbuf)(TAILW)

        slab = slabs[buf]

        def _batch(b, o_, c=c, slab=slab):
            hv = hits_col[pl.ds(b * 16, 16)]
            hp = hits_pos[pl.ds(b * 16, 16)]
            valid = (b * 16 + lanes) < cnt
            m2 = valid & (lax.shift_right_logical(hv, 11) == c)
            ranks = plsc.cumsum(jnp.where(m2, 1, 0).astype(jnp.int32))
            nhit = lax.reduce_max(ranks, (0,))

            flush = o_ + nhit > STAGE_ROWS

            @pl.when(flush)
            def _():
                pltpu.async_copy(stage.at[pl.ds(0, STAGE_ROWS), :],
                                 e_out.at[pos_stage], sem_sc).wait()

            ob = jnp.where(flush, 0, o_)

            @pl.when(nhit > 0)
            def _():
                slots = ranks - 1 + ob
                plsc.store_scatter(pos_stage, [slots & (STAGE_ROWS - 1)],
                                   hp, mask=m2)
                m2i = jnp.where(m2, 1, 0).astype(jnp.int32)
                local = (hv - c * CW) & (CW - 1)
                for j in range(16):
                    idxj = jnp.full((16,), j, jnp.int32)
                    hitj = lax.reduce_max(_vsel(m2i, idxj), (0,))
                    slotv = lax.reduce_max(_vsel(slots, idxj), (0,))
                    colv = lax.reduce_max(_vsel(local, idxj), (0,))
                    sslot = jnp.where(hitj > 0, slotv, STAGE_ROWS + j)
                    row = plsc.load_gather(
                        slab, [lanes, jnp.full((16,), 1, jnp.int32) * colv])
                    plsc.store_scatter(
                        stage, [jnp.full((16,), 1, jnp.int32) * sslot, lanes],
                        row)
            return ob + nhit

        o = lax.fori_loop(0, ngq, _batch, o)

    pltpu.async_copy(stage.at[pl.ds(0, STAGE_ROWS), :],
                     e_out.at[pos_stage], sem_sc).wait()


def _vsel(vec, idxj):
    return lax.gather(
        vec, idxj.reshape(16, 1),
        lax.GatherDimensionNumbers(
            offset_dims=(), collapsed_slice_dims=(0,), start_index_map=(0,)),
        (1,), mode=lax.GatherScatterMode.PROMISE_IN_BOUNDS)


@functools.lru_cache(maxsize=None)
def _sc_gather():
    return pl.kernel(
        _sc_body,
        out_type=(
            jax.ShapeDtypeStruct((EOUT_ROWS, 128), jnp.float32),
            jax.ShapeDtypeStruct((B,), jnp.float32),
        ),
        mesh=plsc.VectorSubcoreMesh(
            core_axis_name="c", subcore_axis_name="s",
            num_cores=NUM_CORES, num_subcores=NUM_SUBCORES,
        ),
        scratch_types=[
            pltpu.VMEM((IPW,), jnp.int32),
            pltpu.VMEM((BPW,), jnp.int32),
            pltpu.VMEM((B,), jnp.int32),
            pltpu.VMEM((B,), jnp.int32),
            pltpu.VMEM((16, CW), jnp.float32),
            pltpu.VMEM((16, CW), jnp.float32),
            pltpu.VMEM((STAGE_ROWS + 16, 128), jnp.float32),
            pltpu.VMEM((STAGE_ROWS,), jnp.int32),
            pltpu.VMEM((BPW,), jnp.float32),
            pltpu.SemaphoreType.DMA,
            pltpu.SemaphoreType.DMA,
            pltpu.SemaphoreType.DMA,
            pltpu.SemaphoreType.DMA,
        ],
        compiler_params=pltpu.CompilerParams(use_tc_tiling_on_sc=True,
                                             needs_layout_passes=False),
    )


def _tc_body(e_ref, wt_ref, g_ref, bb_ref, out_ref):
    e = e_ref[pl.ds(0, B), pl.ds(0, D)]
    wt = wt_ref[...]
    inv_b = 1.0 / B
    ones_b = jnp.ones((B, 1), jnp.float32)
    ones_d = jnp.ones((D, 1), jnp.float32)
    mean_e = lax.dot_general(e, ones_b, (((0,), (0,)), ((), ())),
                             preferred_element_type=jnp.float32) * inv_b
    smom = lax.dot_general(e, e, (((0,), (0,)), ((), ())),
                           preferred_element_type=jnp.float32) * inv_b
    m_t = lax.dot_general(wt, mean_e, (((0,), (0,)), ((), ())))
    p = lax.dot_general(smom, wt, (((1,), (0,)), ((), ())))
    ey2_t = lax.dot_general(wt * p, ones_d, (((0,), (0,)), ((), ())))
    var_t = ey2_t - m_t * m_t
    scale_t = g_ref[...] * lax.rsqrt(var_t + EPS)
    shift_t = bb_ref[...] - m_t * scale_t
    y_t = lax.dot_general(wt, e, (((0,), (1,)), ((), ())),
                          preferred_element_type=jnp.float32)
    out_ref[...] = y_t * scale_t + shift_t


_tc_transform = pl.pallas_call(
    _tc_body,
    out_shape=jax.ShapeDtypeStruct((OUTDIM, B), jnp.float32),
)


@jax.jit
def kernel(x, emb_table, alpha_table, W, b, gamma, beta):
    del b
    alpha_lin = _tc_detile_alpha(alpha_table.T)
    e_pad, alpha = _sc_gather()(emb_table.T, alpha_lin, x)
    y_t = _tc_transform(e_pad, W.T, gamma.reshape(OUTDIM, 1),
                        beta.reshape(OUTDIM, 1))
    return (y_t.T, alpha.reshape(B, 1))


# speedup vs baseline: 5.1204x; 1.4944x over previous
import functools

import jax
import jax.numpy as jnp
from jax import lax
from jax.experimental import pallas as pl
from jax.experimental.pallas import tpu as pltpu
from jax.experimental.pallas import tpu_sc as plsc

B = 16384
D = 16
OUTDIM = 64
NE = 1000000
EPS = 1e-5

NUM_CORES = 2
NUM_SUBCORES = 16
NW = NUM_CORES * NUM_SUBCORES
BPW = B // NW

CW = 2048
TAILW = 640
CPT = 16
IPW = 4096
NPIECE = B // IPW
STAGE_ROWS = 128
EOUT_ROWS = B + STAGE_ROWS


ABLK = 131072
NABLK = (NE + ABLK - 1) // ABLK


def _alpha_body(alpha_ref, alin_ref):
    alin_ref[...] = alpha_ref[0, :]


_tc_detile_alpha = pl.pallas_call(
    _alpha_body,
    grid=(NABLK,),
    in_specs=[pl.BlockSpec((1, ABLK), lambda j: (0, j))],
    out_specs=pl.BlockSpec((ABLK,), lambda j: (j,)),
    out_shape=jax.ShapeDtypeStruct((NABLK * ABLK,), jnp.float32),
)


def _sc_body(emb_hbm, alpha_hbm, idx_hbm, e_out, a_out,
             idx_piece, idx_a, hits_col, hits_pos, slab0, slab1,
             stage, pos_stage, arow_v, sem_a, sem_sl0, sem_sl1, sem_sc):
    wid = lax.axis_index("s") * NUM_CORES + lax.axis_index("c")
    lanes = lax.iota(jnp.int32, 16)
    slabs = [slab0, slab1]
    slab_sems = [sem_sl0, sem_sl1]

    base = wid * BPW
    pltpu.sync_copy(idx_hbm.at[pl.ds(base, BPW)], idx_a)
    cp_a = pltpu.async_copy(alpha_hbm.at[idx_a], arow_v, sem_a)

    def _start_slab(c, buf):
        def go(width):
            pltpu.async_copy(
                emb_hbm.at[pl.ds(0, 8), pl.ds(c * CW, width)],
                slabs[buf].at[pl.ds(0, 8), pl.ds(0, width)], slab_sems[buf])
            pltpu.async_copy(
                emb_hbm.at[pl.ds(8, 8), pl.ds(c * CW, width)],
                slabs[buf].at[pl.ds(8, 8), pl.ds(0, width)], slab_sems[buf])
        return go

    def _wait_slab(c, buf):
        def go(width):
            pltpu.make_async_copy(
                emb_hbm.at[pl.ds(0, 8), pl.ds(c * CW, width)],
                slabs[buf].at[pl.ds(0, 8), pl.ds(0, width)],
                slab_sems[buf]).wait()
            pltpu.make_async_copy(
                emb_hbm.at[pl.ds(8, 8), pl.ds(c * CW, width)],
                slabs[buf].at[pl.ds(8, 8), pl.ds(0, width)],
                slab_sems[buf]).wait()
        return go

    _start_slab(wid, 0)(CW)

    def _prep_piece(piece, off):
        pltpu.sync_copy(idx_hbm.at[pl.ds(piece * IPW, IPW)], idx_piece)

        def _prep_group(g, off_):
            v = idx_piece[pl.ds(g * 16, 16)]
            cid = lax.shift_right_logical(v, 11)
            m = (cid & 31) == wid
            ranks = plsc.cumsum(jnp.where(m, 1, 0).astype(jnp.int32))
            slots = ranks - 1 + off_
            gpos = piece * IPW + g * 16 + lanes
            plsc.store_scatter(hits_col, [slots], v, mask=m)
            plsc.store_scatter(hits_pos, [slots], gpos, mask=m)
            return off_ + lax.reduce_max(ranks, (0,))

        return lax.fori_loop(0, IPW // 16, _prep_group, off)

    cnt = lax.fori_loop(0, NPIECE, _prep_piece, jnp.int32(0))
    cp_a.wait()
    pltpu.sync_copy(arow_v, a_out.at[pl.ds(base, BPW)])

    for g in range(STAGE_ROWS // 16):
        pos_stage[pl.ds(g * 16, 16)] = B + g * 16 + lanes

    ngq = lax.shift_right_logical(cnt + 15, 4)
    o = jnp.int32(0)

    for cc in range(CPT):
        c = cc * 32 + wid
        buf = cc & 1

        @pl.when(c < 488)
        def _(c=c, buf=buf):
            _wait_slab(c, buf)(CW)

        @pl.when(c == 488)
        def _(c=c, buf=buf):
            _wait_slab(c, buf)(TAILW)

        if cc + 1 < CPT:
            cn = (cc + 1) * 32 + wid
            nbuf = (cc + 1) & 1

            @pl.when(cn < 488)
            def _(cn=cn, nbuf=nbuf):
                _start_slab(cn, nbuf)(CW)

            @pl.when(cn == 488)
            def _(cn=cn, nbuf=nbuf):
                _start_slab(cn, nbuf)(TAILW)

        slab = slabs[buf]

        def _batch(b, o_, c=c, slab=slab):
            hv = hits_col[pl.ds(b * 16, 16)]
            hp = hits_pos[pl.ds(b * 16, 16)]
            valid = (b * 16 + lanes) < cnt
            m2 = valid & (lax.shift_right_logical(hv, 11) == c)
            ranks = plsc.cumsum(jnp.where(m2, 1, 0).astype(jnp.int32))
            nhit = lax.reduce_max(ranks, (0,))

            flush = o_ + nhit > STAGE_ROWS

            @pl.when(flush)
            def _():
                pltpu.async_copy(stage, e_out.at[pos_stage], sem_sc).wait()

            ob = jnp.where(flush, 0, o_)

            @pl.when(nhit > 0)
            def _():
                slots = (ranks - 1 + ob) & (STAGE_ROWS - 1)
                plsc.store_scatter(pos_stage, [slots], hp, mask=m2)
                local = (hv - c * CW) & (CW - 1)
                for k in range(16):
                    kvec = jnp.full((16,), k, jnp.int32)
                    vals = plsc.load_gather(slab, [kvec, local])
                    plsc.store_scatter(stage, [slots, kvec], vals, mask=m2)
            return ob + nhit

        o = lax.fori_loop(0, ngq, _batch, o)

    pltpu.async_copy(stage, e_out.at[pos_stage], sem_sc).wait()


@functools.lru_cache(maxsize=None)
def _sc_gather():
    return pl.kernel(
        _sc_body,
        out_type=(
            jax.ShapeDtypeStruct((EOUT_ROWS, 128), jnp.float32),
            jax.ShapeDtypeStruct((B,), jnp.float32),
        ),
        mesh=plsc.VectorSubcoreMesh(
            core_axis_name="c", subcore_axis_name="s",
            num_cores=NUM_CORES, num_subcores=NUM_SUBCORES,
        ),
        scratch_types=[
            pltpu.VMEM((IPW,), jnp.int32),
            pltpu.VMEM((BPW,), jnp.int32),
            pltpu.VMEM((B,), jnp.int32),
            pltpu.VMEM((B,), jnp.int32),
            pltpu.VMEM((16, CW), jnp.float32),
            pltpu.VMEM((16, CW), jnp.float32),
            pltpu.VMEM((STAGE_ROWS, 128), jnp.float32),
            pltpu.VMEM((STAGE_ROWS,), jnp.int32),
            pltpu.VMEM((BPW,), jnp.float32),
            pltpu.SemaphoreType.DMA,
            pltpu.SemaphoreType.DMA,
            pltpu.SemaphoreType.DMA,
            pltpu.SemaphoreType.DMA,
        ],
        compiler_params=pltpu.CompilerParams(use_tc_tiling_on_sc=True,
                                             needs_layout_passes=False),
    )


def _tc_body(e_ref, wt_ref, g_ref, bb_ref, out_ref):
    e = e_ref[pl.ds(0, B), pl.ds(0, D)]
    wt = wt_ref[...]
    inv_b = 1.0 / B
    ones_b = jnp.ones((B, 1), jnp.float32)
    ones_d = jnp.ones((D, 1), jnp.float32)
    mean_e = lax.dot_general(e, ones_b, (((0,), (0,)), ((), ())),
                             preferred_element_type=jnp.float32) * inv_b
    smom = lax.dot_general(e, e, (((0,), (0,)), ((), ())),
                           preferred_element_type=jnp.float32) * inv_b
    m_t = lax.dot_general(wt, mean_e, (((0,), (0,)), ((), ())))
    p = lax.dot_general(smom, wt, (((1,), (0,)), ((), ())))
    ey2_t = lax.dot_general(wt * p, ones_d, (((0,), (0,)), ((), ())))
    var_t = ey2_t - m_t * m_t
    scale_t = g_ref[...] * lax.rsqrt(var_t + EPS)
    shift_t = bb_ref[...] - m_t * scale_t
    y_t = lax.dot_general(wt, e, (((0,), (1,)), ((), ())),
                          preferred_element_type=jnp.float32)
    out_ref[...] = y_t * scale_t + shift_t


_tc_transform = pl.pallas_call(
    _tc_body,
    out_shape=jax.ShapeDtypeStruct((OUTDIM, B), jnp.float32),
)


@jax.jit
def kernel(x, emb_table, alpha_table, W, b, gamma, beta):
    del b
    alpha_lin = _tc_detile_alpha(alpha_table.T)
    e_pad, alpha = _sc_gather()(emb_table.T, alpha_lin, x)
    y_t = _tc_transform(e_pad, W.T, gamma.reshape(OUTDIM, 1),
                        beta.reshape(OUTDIM, 1))
    return (y_t.T, alpha.reshape(B, 1))
